# initial kernel scaffold (unmeasured)
import jax
import jax.numpy as jnp
from jax import lax
from jax.experimental import pallas as pl
from jax.experimental.pallas import tpu as pltpu

N_DEV = 4
B_SH = 64
D = 2048
H_SH = 4096
B = N_DEV * B_SH
KT = 2048
N_T = H_SH // KT


def kernel(x, Win0, Wout0, Win1, Wout1, Win2, Wout2):
    def body(x_ref, win0, wout0, win1, wout1, win2, wout2, out_ref,
             xg, partial, comm_ag, comm_rs, wstage,
             ag_send, ag_recv, rs_send, rs_recv, dma_sems):
        my = lax.axis_index("i")
        left = (my - 1) % N_DEV
        right = (my + 1) % N_DEV

        barrier = pltpu.get_barrier_semaphore()
        for nbr in (left, right):
            pl.semaphore_signal(barrier, inc=1, device_id=(nbr,),
                                device_id_type=pl.DeviceIdType.MESH)
        pl.semaphore_wait(barrier, 2)

        wins = [win0, win1, win2]
        wouts = [wout0, wout1, wout2]

        comm_ag[0] = x_ref[...].astype(jnp.bfloat16)

        for l in range(3):
            xg[pl.ds(my * B_SH, B_SH), :] = comm_ag[0]
            for hp in range(N_DEV - 1):
                rdma = pltpu.make_async_remote_copy(
                    src_ref=comm_ag.at[hp],
                    dst_ref=comm_ag.at[hp + 1],
                    send_sem=ag_send.at[hp],
                    recv_sem=ag_recv.at[hp],
                    device_id=(right,),
                    device_id_type=pl.DeviceIdType.MESH,
                )
                rdma.start()
                rdma.wait()
                origin = (my - hp - 1) % N_DEV
                xg[pl.ds(origin * B_SH, B_SH), :] = comm_ag[hp + 1]

            xg_v = xg[...]
            for j in range(N_T):
                cp_in = pltpu.make_async_copy(
                    wins[l].at[:, pl.ds(j * KT, KT)], wstage.at[0], dma_sems.at[0])
                cp_in.start()
                cp_in.wait()
                h_j = jnp.maximum(
                    jnp.dot(xg_v, wstage[0].astype(jnp.bfloat16),
                            preferred_element_type=jnp.float32),
                    0.0).astype(jnp.bfloat16)
                cp_out = pltpu.make_async_copy(
                    wouts[l].at[pl.ds(j * KT, KT), :], wstage.at[1], dma_sems.at[1])
                cp_out.start()
                cp_out.wait()
                p_j = jnp.dot(h_j, wstage[1].astype(jnp.bfloat16),
                              preferred_element_type=jnp.float32)
                if j == 0:
                    partial[...] = p_j
                else:
                    partial[...] = partial[...] + p_j

            c0 = (my - 1) % N_DEV
            comm_rs[0] = partial[pl.ds(c0 * B_SH, B_SH), :]
            for hp in range(N_DEV - 1):
                rdma = pltpu.make_async_remote_copy(
                    src_ref=comm_rs.at[hp],
                    dst_ref=comm_rs.at[hp + 1],
                    send_sem=rs_send.at[hp],
                    recv_sem=rs_recv.at[hp],
                    device_id=(right,),
                    device_id_type=pl.DeviceIdType.MESH,
                )
                rdma.start()
                rdma.wait()
                c = (my - hp - 2) % N_DEV
                comm_rs[hp + 1] = comm_rs[hp + 1] + partial[pl.ds(c * B_SH, B_SH), :]

            if l < 2:
                comm_ag[0] = comm_rs[N_DEV - 1].astype(jnp.bfloat16)
            else:
                out_ref[...] = comm_rs[N_DEV - 1]

    return pl.pallas_call(
        body,
        out_shape=jax.ShapeDtypeStruct((B_SH, D), jnp.float32),
        in_specs=[
            pl.BlockSpec(memory_space=pltpu.VMEM),
            pl.BlockSpec(memory_space=pltpu.ANY),
            pl.BlockSpec(memory_space=pltpu.ANY),
            pl.BlockSpec(memory_space=pltpu.ANY),
            pl.BlockSpec(memory_space=pltpu.ANY),
            pl.BlockSpec(memory_space=pltpu.ANY),
            pl.BlockSpec(memory_space=pltpu.ANY),
        ],
        out_specs=pl.BlockSpec(memory_space=pltpu.VMEM),
        scratch_shapes=[
            pltpu.VMEM((B, D), jnp.bfloat16),
            pltpu.VMEM((B, D), jnp.float32),
            pltpu.VMEM((N_DEV, B_SH, D), jnp.bfloat16),
            pltpu.VMEM((N_DEV, B_SH, D), jnp.float32),
            pltpu.VMEM((2, D, KT), jnp.float32),
            pltpu.SemaphoreType.DMA((N_DEV - 1,)),
            pltpu.SemaphoreType.DMA((N_DEV - 1,)),
            pltpu.SemaphoreType.DMA((N_DEV - 1,)),
            pltpu.SemaphoreType.DMA((N_DEV - 1,)),
            pltpu.SemaphoreType.DMA((2,)),
        ],
        compiler_params=pltpu.CompilerParams(collective_id=0),
    )(x, Win0, Wout0, Win1, Wout1, Win2, Wout2)


# baseline (device time: 213137 ns/iter reference)
import jax
import jax.numpy as jnp
from jax import lax
from jax.experimental import pallas as pl
from jax.experimental.pallas import tpu as pltpu

N_DEV = 4
B_SH = 64
D = 2048
H_SH = 4096
B = N_DEV * B_SH
KT = 2048
N_T = H_SH // KT


def kernel(x, Win0, Wout0, Win1, Wout1, Win2, Wout2):
    def body(x_ref, win0, wout0, win1, wout1, win2, wout2, out_ref,
             xg, partial, comm_ag, comm_rs, wstage,
             ag_send, ag_recv, rs_send, rs_recv, dma_sems):
        my = lax.axis_index("i")
        left = (my - 1) % N_DEV
        right = (my + 1) % N_DEV

        barrier = pltpu.get_barrier_semaphore()
        for nbr in (left, right):
            pl.semaphore_signal(barrier, inc=1, device_id=(nbr,),
                                device_id_type=pl.DeviceIdType.MESH)
        pl.semaphore_wait(barrier, 2)

        wins = [win0, win1, win2]
        wouts = [wout0, wout1, wout2]

        comm_ag[0] = x_ref[...].astype(jnp.bfloat16)

        for l in range(3):
            xg[pl.ds(my * B_SH, B_SH), :] = comm_ag[0]
            for hp in range(N_DEV - 1):
                rdma = pltpu.make_async_remote_copy(
                    src_ref=comm_ag.at[hp],
                    dst_ref=comm_ag.at[hp + 1],
                    send_sem=ag_send.at[hp],
                    recv_sem=ag_recv.at[hp],
                    device_id=(right,),
                    device_id_type=pl.DeviceIdType.MESH,
                )
                rdma.start()
                rdma.wait()
                origin = (my - hp - 1) % N_DEV
                xg[pl.ds(origin * B_SH, B_SH), :] = comm_ag[hp + 1]

            xg_v = xg[...]
            for j in range(N_T):
                cp_in = pltpu.make_async_copy(
                    wins[l].at[:, pl.ds(j * KT, KT)], wstage.at[0], dma_sems.at[0])
                cp_in.start()
                cp_in.wait()
                h_j = jnp.maximum(
                    jnp.dot(xg_v, wstage[0].astype(jnp.bfloat16),
                            preferred_element_type=jnp.float32),
                    0.0).astype(jnp.bfloat16)
                cp_out = pltpu.make_async_copy(
                    wouts[l].at[pl.ds(j * KT, KT), :], wstage.at[1], dma_sems.at[1])
                cp_out.start()
                cp_out.wait()
                p_j = jnp.dot(h_j, wstage[1].astype(jnp.bfloat16),
                              preferred_element_type=jnp.float32)
                if j == 0:
                    partial[...] = p_j
                else:
                    partial[...] = partial[...] + p_j

            c0 = (my - 1) % N_DEV
            comm_rs[0] = partial[pl.ds(c0 * B_SH, B_SH), :]
            for hp in range(N_DEV - 1):
                rdma = pltpu.make_async_remote_copy(
                    src_ref=comm_rs.at[hp],
                    dst_ref=comm_rs.at[hp + 1],
                    send_sem=rs_send.at[hp],
                    recv_sem=rs_recv.at[hp],
                    device_id=(right,),
                    device_id_type=pl.DeviceIdType.MESH,
                )
                rdma.start()
                rdma.wait()
                c = (my - hp - 2) % N_DEV
                comm_rs[hp + 1] = comm_rs[hp + 1] + partial[pl.ds(c * B_SH, B_SH), :]

            if l < 2:
                comm_ag[0] = comm_rs[N_DEV - 1].astype(jnp.bfloat16)
            else:
                out_ref[...] = comm_rs[N_DEV - 1]

    return pl.pallas_call(
        body,
        out_shape=jax.ShapeDtypeStruct((B_SH, D), jnp.float32),
        in_specs=[
            pl.BlockSpec(memory_space=pltpu.VMEM),
            pl.BlockSpec(memory_space=pltpu.MemorySpace.HBM),
            pl.BlockSpec(memory_space=pltpu.MemorySpace.HBM),
            pl.BlockSpec(memory_space=pltpu.MemorySpace.HBM),
            pl.BlockSpec(memory_space=pltpu.MemorySpace.HBM),
            pl.BlockSpec(memory_space=pltpu.MemorySpace.HBM),
            pl.BlockSpec(memory_space=pltpu.MemorySpace.HBM),
        ],
        out_specs=pl.BlockSpec(memory_space=pltpu.VMEM),
        scratch_shapes=[
            pltpu.VMEM((B, D), jnp.bfloat16),
            pltpu.VMEM((B, D), jnp.float32),
            pltpu.VMEM((N_DEV, B_SH, D), jnp.bfloat16),
            pltpu.VMEM((N_DEV, B_SH, D), jnp.float32),
            pltpu.VMEM((2, D, KT), jnp.float32),
            pltpu.SemaphoreType.DMA((N_DEV - 1,)),
            pltpu.SemaphoreType.DMA((N_DEV - 1,)),
            pltpu.SemaphoreType.DMA((N_DEV - 1,)),
            pltpu.SemaphoreType.DMA((N_DEV - 1,)),
            pltpu.SemaphoreType.DMA((2,)),
        ],
        compiler_params=pltpu.CompilerParams(
            collective_id=0, vmem_limit_bytes=60 * 1024 * 1024),
    )(x, Win0, Wout0, Win1, Wout1, Win2, Wout2)


# device time: 137688 ns/iter; 1.5480x vs baseline; 1.5480x over previous
import jax
import jax.numpy as jnp
from jax import lax
from jax.experimental import pallas as pl
from jax.experimental.pallas import tpu as pltpu

N_DEV = 4
B_SH = 64
D = 2048
H_SH = 4096
B = N_DEV * B_SH
KT = 2048
N_T = H_SH // KT


def kernel(x, Win0, Wout0, Win1, Wout1, Win2, Wout2):
    def body(x_ref, win0, wout0, win1, wout1, win2, wout2, out_ref,
             xg, partial, comm_ag, comm_rs, wstage,
             ag_send, ag_recv, rs_send, rs_recv, dma_sems):
        my = lax.axis_index("i")
        left = (my - 1) % N_DEV
        right = (my + 1) % N_DEV

        barrier = pltpu.get_barrier_semaphore()
        for nbr in (left, right):
            pl.semaphore_signal(barrier, inc=1, device_id=(nbr,),
                                device_id_type=pl.DeviceIdType.MESH)
        pl.semaphore_wait(barrier, 2)

        wins = [win0, win1, win2]
        wouts = [wout0, wout1, wout2]

        def issue_w(l, t, slot):
            if t % 2 == 0:
                src = wins[l].at[:, pl.ds((t // 2) * KT, KT)]
            else:
                src = wouts[l].at[pl.ds((t // 2) * KT, KT), :]
            cp = pltpu.make_async_copy(src, wstage.at[slot], dma_sems.at[slot])
            cp.start()
            return cp

        def wtile(slot):
            return wstage[slot].astype(jnp.bfloat16)

        comm_ag[0] = x_ref[...].astype(jnp.bfloat16)

        cp0 = issue_w(0, 0, 0)
        cp1 = issue_w(0, 1, 1)

        for l in range(3):
            xg[pl.ds(my * B_SH, B_SH), :] = comm_ag[0]
            for hp in range(N_DEV - 1):
                rdma = pltpu.make_async_remote_copy(
                    src_ref=comm_ag.at[hp],
                    dst_ref=comm_ag.at[hp + 1],
                    send_sem=ag_send.at[hp],
                    recv_sem=ag_recv.at[hp],
                    device_id=(right,),
                    device_id_type=pl.DeviceIdType.MESH,
                )
                rdma.start()
                rdma.wait()
                origin = (my - hp - 1) % N_DEV
                xg[pl.ds(origin * B_SH, B_SH), :] = comm_ag[hp + 1]

            xg_v = xg[...]
            cp0.wait()
            h0 = jnp.maximum(
                jnp.dot(xg_v, wtile(0), preferred_element_type=jnp.float32),
                0.0).astype(jnp.bfloat16)
            cp2 = issue_w(l, 2, 0)
            cp1.wait()
            partial[...] = jnp.dot(h0, wtile(1),
                                   preferred_element_type=jnp.float32)
            cp3 = issue_w(l, 3, 1)
            cp2.wait()
            h1 = jnp.maximum(
                jnp.dot(xg_v, wtile(0), preferred_element_type=jnp.float32),
                0.0).astype(jnp.bfloat16)
            cp3.wait()
            partial[...] = partial[...] + jnp.dot(
                h1, wtile(1), preferred_element_type=jnp.float32)

            if l < 2:
                cp0 = issue_w(l + 1, 0, 0)
                cp1 = issue_w(l + 1, 1, 1)

            c0 = (my - 1) % N_DEV
            comm_rs[0] = partial[pl.ds(c0 * B_SH, B_SH), :].astype(jnp.bfloat16)
            result = None
            for hp in range(N_DEV - 1):
                rdma = pltpu.make_async_remote_copy(
                    src_ref=comm_rs.at[hp],
                    dst_ref=comm_rs.at[hp + 1],
                    send_sem=rs_send.at[hp],
                    recv_sem=rs_recv.at[hp],
                    device_id=(right,),
                    device_id_type=pl.DeviceIdType.MESH,
                )
                rdma.start()
                rdma.wait()
                c = (my - hp - 2) % N_DEV
                acc = (comm_rs[hp + 1].astype(jnp.float32)
                       + partial[pl.ds(c * B_SH, B_SH), :])
                if hp < N_DEV - 2:
                    comm_rs[hp + 1] = acc.astype(jnp.bfloat16)
                else:
                    result = acc

            if l < 2:
                comm_ag[0] = result.astype(jnp.bfloat16)
            else:
                out_ref[...] = result

    return pl.pallas_call(
        body,
        out_shape=jax.ShapeDtypeStruct((B_SH, D), jnp.float32),
        in_specs=[
            pl.BlockSpec(memory_space=pltpu.MemorySpace.VMEM),
            pl.BlockSpec(memory_space=pltpu.MemorySpace.HBM),
            pl.BlockSpec(memory_space=pltpu.MemorySpace.HBM),
            pl.BlockSpec(memory_space=pltpu.MemorySpace.HBM),
            pl.BlockSpec(memory_space=pltpu.MemorySpace.HBM),
            pl.BlockSpec(memory_space=pltpu.MemorySpace.HBM),
            pl.BlockSpec(memory_space=pltpu.MemorySpace.HBM),
        ],
        out_specs=pl.BlockSpec(memory_space=pltpu.MemorySpace.VMEM),
        scratch_shapes=[
            pltpu.VMEM((B, D), jnp.bfloat16),
            pltpu.VMEM((B, D), jnp.float32),
            pltpu.VMEM((N_DEV, B_SH, D), jnp.bfloat16),
            pltpu.VMEM((N_DEV, B_SH, D), jnp.bfloat16),
            pltpu.VMEM((2, D, KT), jnp.float32),
            pltpu.SemaphoreType.DMA((N_DEV - 1,)),
            pltpu.SemaphoreType.DMA((N_DEV - 1,)),
            pltpu.SemaphoreType.DMA((N_DEV - 1,)),
            pltpu.SemaphoreType.DMA((N_DEV - 1,)),
            pltpu.SemaphoreType.DMA((2,)),
        ],
        compiler_params=pltpu.CompilerParams(
            collective_id=0, vmem_limit_bytes=60 * 1024 * 1024),
    )(x, Win0, Wout0, Win1, Wout1, Win2, Wout2)


# device time: 111387 ns/iter; 1.9135x vs baseline; 1.2361x over previous
import jax
import jax.numpy as jnp
from jax import lax
from jax.experimental import pallas as pl
from jax.experimental.pallas import tpu as pltpu

N_DEV = 4
B_SH = 64
D = 2048
H_SH = 4096
B = N_DEV * B_SH
KT = 2048
N_T = H_SH // KT


def kernel(x, Win0, Wout0, Win1, Wout1, Win2, Wout2):
    def body(x_ref, win0, wout0, win1, wout1, win2, wout2, out_ref,
             xg, partial, comm_ag, comm_rs, wstage,
             ag_s, ag_r, rs_s, rs_r, dma_sems):
        my = lax.axis_index("i")
        left = (my - 1) % N_DEV
        right = (my + 1) % N_DEV

        barrier = pltpu.get_barrier_semaphore()
        for nbr in (left, right):
            pl.semaphore_signal(barrier, inc=1, device_id=(nbr,),
                                device_id_type=pl.DeviceIdType.MESH)
        pl.semaphore_wait(barrier, 2)

        wins = [win0, win1, win2]
        wouts = [wout0, wout1, wout2]

        def issue_w(l, t, slot):
            if t % 2 == 0:
                src = wins[l].at[:, pl.ds((t // 2) * KT, KT)]
            else:
                src = wouts[l].at[pl.ds((t // 2) * KT, KT), :]
            cp = pltpu.make_async_copy(src, wstage.at[slot], dma_sems.at[slot])
            cp.start()
            return cp

        def wtile(slot):
            return wstage[slot].astype(jnp.bfloat16)

        def pchunk(c):
            return partial[pl.ds(c * B_SH, B_SH), :]

        comm_ag[0] = x_ref[...].astype(jnp.bfloat16)

        cp0 = issue_w(0, 0, 0)
        cp1 = issue_w(0, 1, 1)

        for l in range(3):
            r0R = pltpu.make_async_remote_copy(
                src_ref=comm_ag.at[0], dst_ref=comm_ag.at[1],
                send_sem=ag_s.at[0], recv_sem=ag_r.at[0],
                device_id=(right,), device_id_type=pl.DeviceIdType.MESH)
            r0L = pltpu.make_async_remote_copy(
                src_ref=comm_ag.at[0], dst_ref=comm_ag.at[2],
                send_sem=ag_s.at[1], recv_sem=ag_r.at[1],
                device_id=(left,), device_id_type=pl.DeviceIdType.MESH)
            r0R.start()
            r0L.start()
            xg[pl.ds(my * B_SH, B_SH), :] = comm_ag[0]
            r0R.wait()
            r1 = pltpu.make_async_remote_copy(
                src_ref=comm_ag.at[1], dst_ref=comm_ag.at[3],
                send_sem=ag_s.at[2], recv_sem=ag_r.at[2],
                device_id=(right,), device_id_type=pl.DeviceIdType.MESH)
            r1.start()
            xg[pl.ds(left * B_SH, B_SH), :] = comm_ag[1]
            r0L.wait()
            xg[pl.ds(right * B_SH, B_SH), :] = comm_ag[2]
            r1.wait()
            xg[pl.ds(((my + 2) % N_DEV) * B_SH, B_SH), :] = comm_ag[3]

            xg_v = xg[...]
            cp0.wait()
            h0 = jnp.maximum(
                jnp.dot(xg_v, wtile(0), preferred_element_type=jnp.float32),
                0.0).astype(jnp.bfloat16)
            cp2 = issue_w(l, 2, 0)
            cp1.wait()
            partial[...] = jnp.dot(h0, wtile(1),
                                   preferred_element_type=jnp.float32)
            cp3 = issue_w(l, 3, 1)
            cp2.wait()
            h1 = jnp.maximum(
                jnp.dot(xg_v, wtile(0), preferred_element_type=jnp.float32),
                0.0).astype(jnp.bfloat16)
            cp3.wait()
            partial[...] = partial[...] + jnp.dot(
                h1, wtile(1), preferred_element_type=jnp.float32)

            if l < 2:
                cp0 = issue_w(l + 1, 0, 0)
                cp1 = issue_w(l + 1, 1, 1)

            comm_rs[0] = pchunk((my + 2) % N_DEV).astype(jnp.bfloat16)
            r0 = pltpu.make_async_remote_copy(
                src_ref=comm_rs.at[0], dst_ref=comm_rs.at[1],
                send_sem=rs_s.at[0], recv_sem=rs_r.at[0],
                device_id=(right,), device_id_type=pl.DeviceIdType.MESH)
            r0.start()
            comm_rs[3] = pchunk(left).astype(jnp.bfloat16)
            r0.wait()
            comm_rs[2] = (comm_rs[1].astype(jnp.float32)
                          + pchunk(right)).astype(jnp.bfloat16)
            r1R = pltpu.make_async_remote_copy(
                src_ref=comm_rs.at[2], dst_ref=comm_rs.at[4],
                send_sem=rs_s.at[1], recv_sem=rs_r.at[1],
                device_id=(right,), device_id_type=pl.DeviceIdType.MESH)
            r1L = pltpu.make_async_remote_copy(
                src_ref=comm_rs.at[3], dst_ref=comm_rs.at[5],
                send_sem=rs_s.at[2], recv_sem=rs_r.at[2],
                device_id=(left,), device_id_type=pl.DeviceIdType.MESH)
            r1R.start()
            r1L.start()
            r1R.wait()
            r1L.wait()
            result = (pchunk(my) + comm_rs[4].astype(jnp.float32)
                      + comm_rs[5].astype(jnp.float32))

            if l < 2:
                comm_ag[0] = result.astype(jnp.bfloat16)
            else:
                out_ref[...] = result

    return pl.pallas_call(
        body,
        out_shape=jax.ShapeDtypeStruct((B_SH, D), jnp.float32),
        in_specs=[
            pl.BlockSpec(memory_space=pltpu.MemorySpace.VMEM),
            pl.BlockSpec(memory_space=pltpu.MemorySpace.HBM),
            pl.BlockSpec(memory_space=pltpu.MemorySpace.HBM),
            pl.BlockSpec(memory_space=pltpu.MemorySpace.HBM),
            pl.BlockSpec(memory_space=pltpu.MemorySpace.HBM),
            pl.BlockSpec(memory_space=pltpu.MemorySpace.HBM),
            pl.BlockSpec(memory_space=pltpu.MemorySpace.HBM),
        ],
        out_specs=pl.BlockSpec(memory_space=pltpu.MemorySpace.VMEM),
        scratch_shapes=[
            pltpu.VMEM((B, D), jnp.bfloat16),
            pltpu.VMEM((B, D), jnp.float32),
            pltpu.VMEM((N_DEV, B_SH, D), jnp.bfloat16),
            pltpu.VMEM((6, B_SH, D), jnp.bfloat16),
            pltpu.VMEM((2, D, KT), jnp.float32),
            pltpu.SemaphoreType.DMA((3,)),
            pltpu.SemaphoreType.DMA((3,)),
            pltpu.SemaphoreType.DMA((3,)),
            pltpu.SemaphoreType.DMA((3,)),
            pltpu.SemaphoreType.DMA((2,)),
        ],
        compiler_params=pltpu.CompilerParams(
            collective_id=0, vmem_limit_bytes=60 * 1024 * 1024),
    )(x, Win0, Wout0, Win1, Wout1, Win2, Wout2)
